# baseline (speedup 1.0000x reference)
import functools

import jax
import jax.numpy as jnp
from jax import lax
from jax.experimental import pallas as pl
from jax.experimental.pallas import tpu as pltpu
from jax.experimental.pallas import tpu_sc as plsc

G = 64
NCORE = 2
NSUB = 16
NW = NCORE * NSUB
WIN = 125


def _sc_mesh():
    return plsc.VectorSubcoreMesh(core_axis_name="c", subcore_axis_name="s",
                                  num_cores=NCORE, num_subcores=NSUB)


def _make_deg_kernel(N, E, DW):
    ew = E // NW
    nwin = ew // WIN
    rpt = N // NSUB

    @functools.partial(
        pl.kernel,
        mesh=_sc_mesh(),
        out_type=jax.ShapeDtypeStruct((NW, N // NSUB, DW), jnp.float32),
        scratch_types=[
            pltpu.VMEM((nwin, WIN), jnp.int32),
            pltpu.VMEM((WIN, DW), jnp.float32),
            pltpu.VMEM((WIN, DW), jnp.float32),
            pltpu.VMEM_SHARED((N, DW), jnp.float32),
        ],
    )
    def deg_kernel(dst_hbm, out_hbm, idx_v, ones_v, zeros_v, deg_s):
        cid = lax.axis_index("c")
        sid = lax.axis_index("s")
        wid = cid * NSUB + sid
        nv = DW // 16

        def fill(i, _):
            ones_v[i // nv, pl.ds((i % nv) * 16, 16)] = jnp.ones(
                (16,), jnp.float32)
            zeros_v[i // nv, pl.ds((i % nv) * 16, 16)] = jnp.zeros(
                (16,), jnp.float32)
            return 0

        lax.fori_loop(0, WIN * nv, fill, 0)

        def zero(k, _):
            pltpu.sync_copy(zeros_v, deg_s.at[pl.ds(sid * rpt + k * WIN, WIN)])
            return 0

        lax.fori_loop(0, rpt // WIN, zero, 0)
        plsc.subcore_barrier()

        pltpu.sync_copy(dst_hbm.at[wid], idx_v)

        def body(w, _):
            pltpu.sync_copy(ones_v, deg_s.at[idx_v.at[w]], add=True)
            return 0

        lax.fori_loop(0, nwin, body, 0)
        plsc.subcore_barrier()
        pltpu.sync_copy(deg_s.at[pl.ds(sid * rpt, rpt)], out_hbm.at[wid])

    return deg_kernel


AWIN = 100
ACH = 10
NB = 3


def _make_agg_kernel(N, H, E):
    ew = E // NW
    nwin = ew // AWIN
    nch = nwin // ACH
    rpt = N // NSUB

    @functools.partial(
        pl.kernel,
        mesh=_sc_mesh(),
        out_type=jax.ShapeDtypeStruct((NW, N // NSUB, H), jnp.float32),
        scratch_types=[
            pltpu.VMEM((ACH, AWIN), jnp.int32),
            pltpu.VMEM((ACH, AWIN), jnp.int32),
            pltpu.VMEM((ACH, AWIN), jnp.int32),
            pltpu.VMEM((ACH, AWIN), jnp.int32),
            [pltpu.VMEM((AWIN, H), jnp.float32)] * NB,
            pltpu.VMEM_SHARED((N, H), jnp.float32),
            pltpu.SemaphoreType.DMA,
            pltpu.SemaphoreType.DMA,
            [pltpu.SemaphoreType.DMA] * NB,
        ],
    )
    def agg_kernel(y_hbm, src_hbm, dst_hbm, out_hbm, sbuf0, dbuf0, sbuf1,
                   dbuf1, rows, agg_s, isem0, isem1, gsems):
        cid = lax.axis_index("c")
        sid = lax.axis_index("s")
        wid = cid * NSUB + sid
        ibufs = ((sbuf0, dbuf0, isem0), (sbuf1, dbuf1, isem1))

        def zf(i, _):
            rows[0][i // (H // 16), pl.ds((i % (H // 16)) * 16, 16)] = (
                jnp.zeros((16,), jnp.float32))
            return 0

        lax.fori_loop(0, AWIN * (H // 16), zf, 0)

        nz = rpt // AWIN
        def zero(k, _):
            pltpu.sync_copy(rows[0], agg_s.at[pl.ds(sid * rpt + k * AWIN, AWIN)])
            return 0

        lax.fori_loop(0, nz, zero, 0)
        if rpt % AWIN:
            r = rpt % AWIN
            pltpu.sync_copy(rows[0].at[pl.ds(0, r)],
                            agg_s.at[pl.ds(sid * rpt + nz * AWIN, r)])
        plsc.subcore_barrier()

        pltpu.async_copy(src_hbm.at[wid * nch], sbuf0, isem0)
        pltpu.async_copy(dst_hbm.at[wid * nch], dbuf0, isem0)

        for c in range(nch):
            sb, db, isem = ibufs[c % 2]
            pltpu.make_async_copy(src_hbm.at[wid * nch + c], sb, isem).wait()
            pltpu.make_async_copy(dst_hbm.at[wid * nch + c], db, isem).wait()
            if c + 1 < nch:
                nsb, ndb, nisem = ibufs[(c + 1) % 2]
                pltpu.async_copy(src_hbm.at[wid * nch + c + 1], nsb, nisem)
                pltpu.async_copy(dst_hbm.at[wid * nch + c + 1], ndb, nisem)

            for b in range(NB - 1):
                pltpu.async_copy(y_hbm.at[sb.at[b]], rows[b], gsems[b])

            def body(wg, _, sb=sb, db=db):
                for b in range(NB):
                    w = wg * NB + b
                    rb, gs = rows[b], gsems[b]
                    pltpu.make_async_copy(y_hbm.at[sb.at[w]], rb, gs).wait()

                    nb_ = (b + NB - 1) % NB

                    @pl.when(w + NB - 1 < ACH)
                    def _():
                        pltpu.async_copy(y_hbm.at[sb.at[w + NB - 1]],
                                         rows[nb_], gsems[nb_])

                    pltpu.sync_copy(rb, agg_s.at[db.at[w]], add=True)
                return 0

            lax.fori_loop(0, ACH // NB, body, 0)
        plsc.subcore_barrier()
        pltpu.sync_copy(agg_s.at[pl.ds(sid * rpt, rpt)], out_hbm.at[wid])

    return agg_kernel


DW = 128


def _dinv_from(dp0, dp1):
    deg = (jnp.sum(dp0, axis=1) + jnp.sum(dp1, axis=1)) * (1.0 / DW) + 1.0
    return lax.rsqrt(deg)[:, None]


def _k1a_body(x_ref, w1_ref, xw_ref):
    xw_ref[...] = jnp.dot(x_ref[...], w1_ref[...],
                          preferred_element_type=jnp.float32)


def _k1b_body(xw_ref, dp0_ref, dp1_ref, y1_ref, dv_ref):
    dv = _dinv_from(dp0_ref[0], dp1_ref[0])
    dv_ref[0, 0] = dv[:, 0]
    y1_ref[...] = xw_ref[...] * dv


def _k2_body(a0_ref, a1_ref, y1_ref, dv_ref, w2_ref, b1_ref, y2_ref):
    dinv = dv_ref[0, 0][:, None]
    h1 = jax.nn.relu(dinv * (a0_ref[...] + a1_ref[...] + y1_ref[...])
                     + b1_ref[...])
    y2_ref[...] = jnp.dot(h1, w2_ref[...],
                          preferred_element_type=jnp.float32) * dinv


def _k3_body(a0_ref, a1_ref, y2_ref, dv_ref, b2_ref, batch_ref,
             wfc_ref, bfc_ref, out_ref, sacc, cacc, *, nsteps, bs):
    i = pl.program_id(0)
    dinv = dv_ref[0, 0][:, None]
    h2 = dinv * (a0_ref[...] + a1_ref[...] + y2_ref[...]) + b2_ref[...]
    seg = (lax.broadcasted_iota(jnp.int32, (G, bs), 0)
           == batch_ref[0, 0][None, :]).astype(jnp.float32)
    part = jnp.dot(seg, h2, preferred_element_type=jnp.float32)
    cnt = jnp.sum(seg, axis=1, keepdims=True) * jnp.ones((1, h2.shape[1]),
                                                         jnp.float32)

    @pl.when(i == 0)
    def _():
        sacc[...] = part
        cacc[...] = cnt

    @pl.when(i > 0)
    def _():
        sacc[...] += part
        cacc[...] += cnt

    @pl.when(i == nsteps - 1)
    def _():
        pooled = sacc[...] / jnp.maximum(cacc[...], 1.0)
        logits = jnp.dot(pooled, wfc_ref[...],
                         preferred_element_type=jnp.float32) + bfc_ref[...]
        m = jnp.max(logits, axis=1, keepdims=True)
        ls = logits - m
        out_ref[...] = ls - jnp.log(jnp.sum(jnp.exp(ls), axis=1,
                                            keepdims=True))


def kernel(x, edge_index, batch, W1, b1, W2, b2, Wfc, bfc):
    N, D = x.shape
    H = W1.shape[1]
    C = Wfc.shape[1]
    E = edge_index.shape[1]

    nwin = E // NW // AWIN
    src = edge_index[0].reshape(NW * nwin // ACH, ACH, AWIN)
    dst = edge_index[1].reshape(NW * nwin // ACH, ACH, AWIN)
    dst_deg = edge_index[1].reshape(NW, E // NW // WIN, WIN)

    deg_kernel = _make_deg_kernel(N, E, DW)
    agg_kernel = _make_agg_kernel(N, H, E)

    degp = deg_kernel(dst_deg).reshape(NCORE, N, DW)

    BS = 1000
    nsteps = N // BS
    full = lambda *shape: pl.BlockSpec(shape, lambda i: (0,) * len(shape))
    rows = lambda *rest: pl.BlockSpec((BS,) + rest,
                                      lambda i: (i,) + (0,) * len(rest))
    dp_spec0 = pl.BlockSpec((1, BS, DW), lambda i: (0, i, 0))
    dp_spec1 = pl.BlockSpec((1, BS, DW), lambda i: (1, i, 0))

    dv_spec = pl.BlockSpec((1, 1, BS), lambda i: (i, 0, 0))
    xw = pl.pallas_call(
        _k1a_body,
        grid=(nsteps,),
        in_specs=[rows(D), full(D, H)],
        out_specs=rows(H),
        out_shape=jax.ShapeDtypeStruct((N, H), jnp.float32),
    )(x, W1)

    y1, dv = pl.pallas_call(
        _k1b_body,
        grid=(nsteps,),
        in_specs=[rows(H), dp_spec0, dp_spec1],
        out_specs=[rows(H), dv_spec],
        out_shape=[jax.ShapeDtypeStruct((N, H), jnp.float32),
                   jax.ShapeDtypeStruct((nsteps, 1, BS), jnp.float32)],
    )(xw, degp, degp)

    agg1 = agg_kernel(y1, src, dst).reshape(NCORE, N, H)

    a_spec0 = pl.BlockSpec((1, BS, H), lambda i: (0, i, 0))
    a_spec1 = pl.BlockSpec((1, BS, H), lambda i: (1, i, 0))

    def _k2(a0, a1, y1r, dvr, w2, b1r, y2r):
        _k2_body(a0.at[0], a1.at[0], y1r, dvr, w2, b1r, y2r)

    y2 = pl.pallas_call(
        _k2,
        grid=(nsteps,),
        in_specs=[a_spec0, a_spec1, rows(H), dv_spec,
                  full(H, H), full(1, H)],
        out_specs=rows(H),
        out_shape=jax.ShapeDtypeStruct((N, H), jnp.float32),
    )(agg1, agg1, y1, dv, W2, b1.reshape(1, H))

    agg2 = agg_kernel(y2, src, dst).reshape(NCORE, N, H)

    def _k3(a0, a1, y2r, dvr, b2r, batch_r, wfc, bfc_r, out, sacc, cacc):
        _k3_body(a0.at[0], a1.at[0], y2r, dvr, b2r, batch_r, wfc, bfc_r,
                 out, sacc, cacc, nsteps=nsteps, bs=BS)

    out = pl.pallas_call(
        _k3,
        grid=(nsteps,),
        in_specs=[a_spec0, a_spec1, rows(H), dv_spec, full(1, H),
                  pl.BlockSpec((1, 1, BS), lambda i: (i, 0, 0)),
                  full(H, C), full(1, C)],
        out_specs=full(G, C),
        out_shape=jax.ShapeDtypeStruct((G, C), jnp.float32),
        scratch_shapes=[pltpu.VMEM((G, H), jnp.float32),
                        pltpu.VMEM((G, H), jnp.float32)],
    )(agg2, agg2, y2, dv, b2.reshape(1, H),
      batch.reshape(nsteps, 1, BS), Wfc, bfc.reshape(1, C))

    return out


# speedup vs baseline: 1.0747x; 1.0747x over previous
import functools

import jax
import jax.numpy as jnp
from jax import lax
from jax.experimental import pallas as pl
from jax.experimental.pallas import tpu as pltpu
from jax.experimental.pallas import tpu_sc as plsc

G = 64
NCORE = 2
NSUB = 16
NW = NCORE * NSUB
WIN = 125


def _sc_mesh():
    return plsc.VectorSubcoreMesh(core_axis_name="c", subcore_axis_name="s",
                                  num_cores=NCORE, num_subcores=NSUB)


RPT = 640


def _tile_start(sid, N):
    return jnp.minimum(sid * RPT, N - RPT)


def _make_deg_kernel(N, E, DW):
    ew = E // NW
    nwin = ew // WIN

    @functools.partial(
        pl.kernel,
        mesh=_sc_mesh(),
        out_type=jax.ShapeDtypeStruct((NCORE * N, DW), jnp.float32),
        scratch_types=[
            pltpu.VMEM((nwin, WIN), jnp.int32),
            pltpu.VMEM((WIN, DW), jnp.float32),
            pltpu.VMEM((WIN, DW), jnp.float32),
            pltpu.VMEM_SHARED((N, DW), jnp.float32),
        ],
    )
    def deg_kernel(dst_hbm, out_hbm, idx_v, ones_v, zeros_v, deg_s):
        cid = lax.axis_index("c")
        sid = lax.axis_index("s")
        wid = cid * NSUB + sid
        nv = DW // 16
        start = _tile_start(sid, N)

        def fill(i, _):
            ones_v[i // nv, pl.ds((i % nv) * 16, 16)] = jnp.ones(
                (16,), jnp.float32)
            zeros_v[i // nv, pl.ds((i % nv) * 16, 16)] = jnp.zeros(
                (16,), jnp.float32)
            return 0

        lax.fori_loop(0, WIN * nv, fill, 0)

        def zero(k, _):
            pltpu.sync_copy(zeros_v, deg_s.at[pl.ds(start + k * WIN, WIN)])
            return 0

        lax.fori_loop(0, RPT // WIN, zero, 0)
        if RPT % WIN:
            r = RPT % WIN
            pltpu.sync_copy(zeros_v.at[pl.ds(0, r)],
                            deg_s.at[pl.ds(start + (RPT // WIN) * WIN, r)])
        plsc.subcore_barrier()

        pltpu.sync_copy(dst_hbm.at[wid], idx_v)

        def body(w, _):
            pltpu.sync_copy(ones_v, deg_s.at[idx_v.at[w]], add=True)
            return 0

        lax.fori_loop(0, nwin, body, 0)
        plsc.subcore_barrier()
        pltpu.sync_copy(deg_s.at[pl.ds(start, RPT)],
                        out_hbm.at[pl.ds(cid * N + start, RPT)])

    return deg_kernel


AWIN = 100
ACH = 10
NB = 3


def _make_agg_kernel(N, H, E):
    ew = E // NW
    nwin = ew // AWIN
    nch = nwin // ACH

    @functools.partial(
        pl.kernel,
        mesh=_sc_mesh(),
        out_type=jax.ShapeDtypeStruct((NCORE * N, H), jnp.float32),
        scratch_types=[
            pltpu.VMEM((ACH, AWIN), jnp.int32),
            pltpu.VMEM((ACH, AWIN), jnp.int32),
            pltpu.VMEM((ACH, AWIN), jnp.int32),
            pltpu.VMEM((ACH, AWIN), jnp.int32),
            [pltpu.VMEM((AWIN, H), jnp.float32)] * NB,
            pltpu.VMEM_SHARED((N, H), jnp.float32),
            pltpu.SemaphoreType.DMA,
            pltpu.SemaphoreType.DMA,
            [pltpu.SemaphoreType.DMA] * NB,
        ],
    )
    def agg_kernel(y_hbm, src_hbm, dst_hbm, out_hbm, sbuf0, dbuf0, sbuf1,
                   dbuf1, rows, agg_s, isem0, isem1, gsems):
        cid = lax.axis_index("c")
        sid = lax.axis_index("s")
        wid = cid * NSUB + sid
        ibufs = ((sbuf0, dbuf0, isem0), (sbuf1, dbuf1, isem1))
        start = _tile_start(sid, N)

        def zf(i, _):
            rows[0][i // (H // 16), pl.ds((i % (H // 16)) * 16, 16)] = (
                jnp.zeros((16,), jnp.float32))
            return 0

        lax.fori_loop(0, AWIN * (H // 16), zf, 0)

        nz = RPT // AWIN
        def zero(k, _):
            pltpu.sync_copy(rows[0], agg_s.at[pl.ds(start + k * AWIN, AWIN)])
            return 0

        lax.fori_loop(0, nz, zero, 0)
        if RPT % AWIN:
            r = RPT % AWIN
            pltpu.sync_copy(rows[0].at[pl.ds(0, r)],
                            agg_s.at[pl.ds(start + nz * AWIN, r)])
        plsc.subcore_barrier()

        pltpu.async_copy(src_hbm.at[wid * nch], sbuf0, isem0)
        pltpu.async_copy(dst_hbm.at[wid * nch], dbuf0, isem0)

        for c in range(nch):
            sb, db, isem = ibufs[c % 2]
            pltpu.make_async_copy(src_hbm.at[wid * nch + c], sb, isem).wait()
            pltpu.make_async_copy(dst_hbm.at[wid * nch + c], db, isem).wait()
            if c + 1 < nch:
                nsb, ndb, nisem = ibufs[(c + 1) % 2]
                pltpu.async_copy(src_hbm.at[wid * nch + c + 1], nsb, nisem)
                pltpu.async_copy(dst_hbm.at[wid * nch + c + 1], ndb, nisem)

            for b in range(NB - 1):
                pltpu.async_copy(y_hbm.at[sb.at[b]], rows[b], gsems[b])

            def body(wg, _, sb=sb, db=db):
                for b in range(NB):
                    w = wg * NB + b
                    rb, gs = rows[b], gsems[b]
                    pltpu.make_async_copy(y_hbm.at[sb.at[w]], rb, gs).wait()

                    nb_ = (b + NB - 1) % NB

                    @pl.when(w + NB - 1 < ACH)
                    def _():
                        pltpu.async_copy(y_hbm.at[sb.at[w + NB - 1]],
                                         rows[nb_], gsems[nb_])

                    pltpu.sync_copy(rb, agg_s.at[db.at[w]], add=True)
                return 0

            lax.fori_loop(0, ACH // NB, body, 0)
        plsc.subcore_barrier()
        pltpu.sync_copy(agg_s.at[pl.ds(start, RPT)],
                        out_hbm.at[pl.ds(cid * N + start, RPT)])

    return agg_kernel


DW = 128


def _dinv_from(dp0, dp1):
    deg = (jnp.sum(dp0, axis=1) + jnp.sum(dp1, axis=1)) * (1.0 / DW) + 1.0
    return lax.rsqrt(deg)[:, None]


def _k1a_body(x_ref, w1_ref, xw_ref):
    xw_ref[...] = jnp.dot(x_ref[...], w1_ref[...],
                          preferred_element_type=jnp.float32)


def _k1b_body(xw_ref, dp0_ref, dp1_ref, y1_ref, dv_ref):
    dv = _dinv_from(dp0_ref[...], dp1_ref[...])
    dv_ref[0, 0] = dv[:, 0]
    y1_ref[...] = xw_ref[...] * dv


def _k2_body(a0_ref, a1_ref, y1_ref, dv_ref, w2_ref, b1_ref, y2_ref):
    dinv = dv_ref[0, 0][:, None]
    h1 = jax.nn.relu(dinv * (a0_ref[...] + a1_ref[...] + y1_ref[...])
                     + b1_ref[...])
    y2_ref[...] = jnp.dot(h1, w2_ref[...],
                          preferred_element_type=jnp.float32) * dinv


def _k3_body(a0_ref, a1_ref, y2_ref, dv_ref, b2_ref, batch_ref,
             wfc_ref, bfc_ref, out_ref, sacc, cacc, *, nsteps, bs):
    i = pl.program_id(0)
    dinv = dv_ref[0, 0][:, None]
    h2 = dinv * (a0_ref[...] + a1_ref[...] + y2_ref[...]) + b2_ref[...]
    seg = (lax.broadcasted_iota(jnp.int32, (G, bs), 0)
           == batch_ref[0, 0][None, :]).astype(jnp.float32)
    part = jnp.dot(seg, h2, preferred_element_type=jnp.float32)
    cnt = jnp.sum(seg, axis=1, keepdims=True) * jnp.ones((1, h2.shape[1]),
                                                         jnp.float32)

    @pl.when(i == 0)
    def _():
        sacc[...] = part
        cacc[...] = cnt

    @pl.when(i > 0)
    def _():
        sacc[...] += part
        cacc[...] += cnt

    @pl.when(i == nsteps - 1)
    def _():
        pooled = sacc[...] / jnp.maximum(cacc[...], 1.0)
        logits = jnp.dot(pooled, wfc_ref[...],
                         preferred_element_type=jnp.float32) + bfc_ref[...]
        m = jnp.max(logits, axis=1, keepdims=True)
        ls = logits - m
        out_ref[...] = ls - jnp.log(jnp.sum(jnp.exp(ls), axis=1,
                                            keepdims=True))


def kernel(x, edge_index, batch, W1, b1, W2, b2, Wfc, bfc):
    N, D = x.shape
    H = W1.shape[1]
    C = Wfc.shape[1]
    E = edge_index.shape[1]

    nwin = E // NW // AWIN
    src = edge_index[0].reshape(NW * nwin // ACH, ACH, AWIN)
    dst = edge_index[1].reshape(NW * nwin // ACH, ACH, AWIN)
    dst_deg = edge_index[1].reshape(NW, E // NW // WIN, WIN)

    deg_kernel = _make_deg_kernel(N, E, DW)
    agg_kernel = _make_agg_kernel(N, H, E)

    degp = deg_kernel(dst_deg)

    BS = 1000
    nsteps = N // BS
    full = lambda *shape: pl.BlockSpec(shape, lambda i: (0,) * len(shape))
    rows = lambda *rest: pl.BlockSpec((BS,) + rest,
                                      lambda i: (i,) + (0,) * len(rest))
    dp_spec0 = pl.BlockSpec((BS, DW), lambda i: (i, 0))
    dp_spec1 = pl.BlockSpec((BS, DW), lambda i: (i + nsteps, 0))

    dv_spec = pl.BlockSpec((1, 1, BS), lambda i: (i, 0, 0))
    xw = pl.pallas_call(
        _k1a_body,
        grid=(nsteps,),
        in_specs=[rows(D), full(D, H)],
        out_specs=rows(H),
        out_shape=jax.ShapeDtypeStruct((N, H), jnp.float32),
    )(x, W1)

    y1, dv = pl.pallas_call(
        _k1b_body,
        grid=(nsteps,),
        in_specs=[rows(H), dp_spec0, dp_spec1],
        out_specs=[rows(H), dv_spec],
        out_shape=[jax.ShapeDtypeStruct((N, H), jnp.float32),
                   jax.ShapeDtypeStruct((nsteps, 1, BS), jnp.float32)],
    )(xw, degp, degp)

    agg1 = agg_kernel(y1, src, dst)

    a_spec0 = pl.BlockSpec((BS, H), lambda i: (i, 0))
    a_spec1 = pl.BlockSpec((BS, H), lambda i: (i + nsteps, 0))

    y2 = pl.pallas_call(
        _k2_body,
        grid=(nsteps,),
        in_specs=[a_spec0, a_spec1, rows(H), dv_spec,
                  full(H, H), full(1, H)],
        out_specs=rows(H),
        out_shape=jax.ShapeDtypeStruct((N, H), jnp.float32),
    )(agg1, agg1, y1, dv, W2, b1.reshape(1, H))

    agg2 = agg_kernel(y2, src, dst)

    _k3 = functools.partial(_k3_body, nsteps=nsteps, bs=BS)

    out = pl.pallas_call(
        _k3,
        grid=(nsteps,),
        in_specs=[a_spec0, a_spec1, rows(H), dv_spec, full(1, H),
                  pl.BlockSpec((1, 1, BS), lambda i: (i, 0, 0)),
                  full(H, C), full(1, C)],
        out_specs=full(G, C),
        out_shape=jax.ShapeDtypeStruct((G, C), jnp.float32),
        scratch_shapes=[pltpu.VMEM((G, H), jnp.float32),
                        pltpu.VMEM((G, H), jnp.float32)],
    )(agg2, agg2, y2, dv, b2.reshape(1, H),
      batch.reshape(nsteps, 1, BS), Wfc, bfc.reshape(1, C))

    return out
